# baseline (speedup 1.0000x reference)
import functools

import jax
import jax.numpy as jnp
from jax import lax
from jax.experimental import pallas as pl
from jax.experimental.pallas import tpu as pltpu
from jax.experimental.pallas import tpu_sc as plsc

N_NODES = 10000
N_EDGES = 320000
D = 128
RRELU_SLOPE = (1.0 / 8.0 + 1.0 / 3.0) / 2.0

NUM_CORES = 2
NUM_SUBCORES = 16
TILE_ROW_BASE = 624
TILE_ROW_SPAN = 640
UNIT = 128
N_UNITS = N_EDGES // UNIT
BATCH = 16
PAD_ROWS = 2512


def _sc_body(src2d, dst2d, nf, ef, g_out, e_out, cnt_out,
             sidxb, didxb, rows, ones, zbuf, acc, cnt_sh,
             semf0, semf1, sems, semc):
  cid = lax.axis_index("c")
  sid = lax.axis_index("s")

  zero16 = jnp.zeros((16,), jnp.float32)
  one16 = jnp.ones((16,), jnp.float32)

  def _zero_rows(i, carry):
    for k in range(8):
      rows[0, i, pl.ds(k * 16, 16)] = zero16
    return carry
  lax.fori_loop(0, UNIT, _zero_rows, 0)

  def _zero_z(i, carry):
    zbuf[pl.ds(i * 16, 16)] = zero16
    return carry
  lax.fori_loop(0, TILE_ROW_SPAN // 16, _zero_z, 0)

  for k in range(8):
    ones[pl.ds(k * 16, 16)] = one16

  abase = sid * TILE_ROW_BASE
  zdescs = [pltpu.async_copy(rows.at[0, pl.ds(0, UNIT)],
                             acc.at[pl.ds(abase + off, UNIT)], sems)
            for off in range(0, TILE_ROW_SPAN, UNIT)]
  zdescs.append(pltpu.async_copy(zbuf,
                                 cnt_sh.at[pl.ds(abase, TILE_ROW_SPAN)],
                                 sems))
  for zd in zdescs:
    zd.wait()

  plsc.subcore_barrier()

  u0 = (N_UNITS * sid) // NUM_SUBCORES
  u1 = (N_UNITS * (sid + 1)) // NUM_SUBCORES
  n_units = u1 - u0

  def _drain_cnt():
    pltpu.make_async_copy(zbuf.at[pl.ds(0, UNIT)],
                          cnt_sh.at[pl.ds(0, UNIT)], semc).wait()

  def _drain_rows(sem):
    pltpu.make_async_copy(rows.at[0], acc.at[pl.ds(0, UNIT)], sem).wait()

  def _wait_fill(i):
    @pl.when(lax.rem(i, 2) == 0)
    def _():
      _drain_rows(semf0)

    @pl.when(lax.rem(i, 2) == 1)
    def _():
      _drain_rows(semf1)

  def _issue_scatter(i):
    slot = lax.rem(i, 2)
    p = lax.rem(i // BATCH, 2)
    r = lax.rem(i, BATCH)

    pltpu.async_copy(rows.at[slot], acc.at[didxb.at[p, r]], sems,
                     add=True)

    @pl.when(cid == 1)
    def _():
      pltpu.async_copy(ones, cnt_sh.at[didxb.at[p, r]], semc, add=True)

  def _unit(i, carry):
    slot = lax.rem(i, 2)
    p = lax.rem(i // BATCH, 2)
    r = lax.rem(i, BATCH)
    u = u0 + i

    @pl.when(i > 1)
    def _():
      _drain_rows(sems)

      @pl.when(cid == 1)
      def _():
        _drain_cnt()

    @pl.when(r == 0)
    def _():
      pltpu.sync_copy(dst2d.at[pl.ds(u, BATCH)], didxb.at[p])

      @pl.when(cid == 0)
      def _():
        pltpu.sync_copy(src2d.at[pl.ds(u, BATCH)], sidxb.at[p])

    @pl.when(jnp.logical_and(cid == 0, slot == 0))
    def _():
      pltpu.async_copy(nf.at[sidxb.at[p, r]], rows.at[0], semf0)

    @pl.when(jnp.logical_and(cid == 0, slot == 1))
    def _():
      pltpu.async_copy(nf.at[sidxb.at[p, r]], rows.at[1], semf1)

    @pl.when(jnp.logical_and(cid == 1, slot == 0))
    def _():
      pltpu.async_copy(ef.at[pl.ds(u * UNIT, UNIT)], rows.at[0], semf0)

    @pl.when(jnp.logical_and(cid == 1, slot == 1))
    def _():
      pltpu.async_copy(ef.at[pl.ds(u * UNIT, UNIT)], rows.at[1], semf1)

    @pl.when(i > 0)
    def _():
      _wait_fill(i - 1)
      _issue_scatter(i - 1)

    return carry

  lax.fori_loop(0, n_units, _unit, 0)

  _wait_fill(n_units - 1)
  _issue_scatter(n_units - 1)
  _drain_rows(sems)
  _drain_rows(sems)

  @pl.when(cid == 1)
  def _():
    _drain_cnt()
    _drain_cnt()

  plsc.subcore_barrier()

  @pl.when(cid == 0)
  def _out_g():
    pltpu.sync_copy(acc.at[pl.ds(abase, TILE_ROW_SPAN)],
                    g_out.at[pl.ds(abase, TILE_ROW_SPAN)])

  @pl.when(cid == 1)
  def _out_e():
    pltpu.sync_copy(acc.at[pl.ds(abase, TILE_ROW_SPAN)],
                    e_out.at[pl.ds(abase, TILE_ROW_SPAN)])
    pltpu.sync_copy(cnt_sh.at[pl.ds(abase, TILE_ROW_SPAN)], zbuf)
    pltpu.sync_copy(zbuf, cnt_out.at[pl.ds(abase, TILE_ROW_SPAN)])


def _segment_sums(src2d, dst2d, node_feats, edge_feats):
  mesh = plsc.VectorSubcoreMesh(
      core_axis_name="c", subcore_axis_name="s",
      num_cores=NUM_CORES, num_subcores=NUM_SUBCORES)
  f = pl.kernel(
      _sc_body,
      out_type=[
          jax.ShapeDtypeStruct((N_NODES, D), jnp.float32),
          jax.ShapeDtypeStruct((N_NODES, D), jnp.float32),
          jax.ShapeDtypeStruct((N_NODES,), jnp.float32),
      ],
      mesh=mesh,
      scratch_types=[
          pltpu.VMEM((2, BATCH, 128), jnp.int32),
          pltpu.VMEM((2, BATCH, 128), jnp.int32),
          pltpu.VMEM((2, UNIT, D), jnp.float32),
          pltpu.VMEM((128,), jnp.float32),
          pltpu.VMEM((TILE_ROW_SPAN,), jnp.float32),
          pltpu.VMEM_SHARED((N_NODES, D), jnp.float32),
          pltpu.VMEM_SHARED((N_NODES,), jnp.float32),
          pltpu.SemaphoreType.DMA,
          pltpu.SemaphoreType.DMA,
          pltpu.SemaphoreType.DMA,
          pltpu.SemaphoreType.DMA,
      ],
      compiler_params=pltpu.CompilerParams(use_tc_tiling_on_sc=False),
  )
  return f(src2d, dst2d, node_feats, edge_feats)


def _tail_body(g_ref, e_ref, cntp_ref, w1a_ref, w1b_ref, w2_ref,
               b1_ref, b2_ref, out_ref):
  cnt = cntp_ref[...][:, 0]
  ms = (jnp.dot(g_ref[...], w1a_ref[...], preferred_element_type=jnp.float32)
        + jnp.dot(e_ref[...], w1b_ref[...], preferred_element_type=jnp.float32)
        + cnt[:, None] * b1_ref[...])
  nm = ms / jnp.maximum(cnt, 1.0)[:, None]
  sm = jnp.dot(nm, w2_ref[...], preferred_element_type=jnp.float32) + b2_ref[...]
  o = nm + sm
  out_ref[...] = jnp.where(o >= 0, o, o * RRELU_SLOPE)


def _tail(g, e, cntp, w1a_t, w1b_t, w2_t, b1, b2):
  blk = 2000
  grid = (N_NODES // blk,)
  return pl.pallas_call(
      _tail_body,
      grid=grid,
      in_specs=[
          pl.BlockSpec((blk, D), lambda i: (i, 0)),
          pl.BlockSpec((blk, D), lambda i: (i, 0)),
          pl.BlockSpec((blk, 1), lambda i: (i, 0)),
          pl.BlockSpec((D, D), lambda i: (0, 0)),
          pl.BlockSpec((D, D), lambda i: (0, 0)),
          pl.BlockSpec((D, D), lambda i: (0, 0)),
          pl.BlockSpec((1, D), lambda i: (0, 0)),
          pl.BlockSpec((1, D), lambda i: (0, 0)),
      ],
      out_specs=pl.BlockSpec((blk, D), lambda i: (i, 0)),
      out_shape=jax.ShapeDtypeStruct((N_NODES, D), jnp.float32),
  )(g, e, cntp, w1a_t, w1b_t, w2_t, b1, b2)


@jax.jit
def kernel(node_feats, edge_feats, edge_index, W1, b1, W2, b2):
  pad = PAD_ROWS * 128 - N_EDGES
  src2d = jnp.pad(edge_index[0].astype(jnp.int32),
                  (0, pad)).reshape(PAD_ROWS, 128)
  dst2d = jnp.pad(edge_index[1].astype(jnp.int32),
                  (0, pad)).reshape(PAD_ROWS, 128)
  g, e, cnt = _segment_sums(src2d, dst2d, node_feats, edge_feats)
  cntp = cnt.reshape(N_NODES, 1)
  w1a_t = W1[:, :D].T
  w1b_t = W1[:, D:].T
  w2_t = W2.T
  return _tail(g, e, cntp, w1a_t, w1b_t, w2_t,
               b1.reshape(1, D), b2.reshape(1, D))


# speedup vs baseline: 1.0193x; 1.0193x over previous
import functools

import jax
import jax.numpy as jnp
from jax import lax
from jax.experimental import pallas as pl
from jax.experimental.pallas import tpu as pltpu
from jax.experimental.pallas import tpu_sc as plsc

N_NODES = 10000
N_EDGES = 320000
D = 128
RRELU_SLOPE = (1.0 / 8.0 + 1.0 / 3.0) / 2.0

NUM_CORES = 2
NUM_SUBCORES = 16
TILE_ROW_BASE = 624
TILE_ROW_SPAN = 640
UNIT = 128
N_UNITS = N_EDGES // UNIT
BATCH = 16
PAD_ROWS = 2512


def _sc_body(src2d, dst2d, nf, ef, g_out, e_out, cnt_out,
             sidxb, didxb, rows, ones, zbuf, acc, cnt_sh,
             semf0, semf1, sems, semc):
  cid = lax.axis_index("c")
  sid = lax.axis_index("s")

  zero16 = jnp.zeros((16,), jnp.float32)
  one16 = jnp.ones((16,), jnp.float32)

  def _zero_rows(i, carry):
    for k in range(8):
      rows[0, i, pl.ds(k * 16, 16)] = zero16
    return carry
  lax.fori_loop(0, UNIT, _zero_rows, 0)

  def _zero_z(i, carry):
    zbuf[pl.ds(i * 16, 16)] = zero16
    return carry
  lax.fori_loop(0, TILE_ROW_SPAN // 16, _zero_z, 0)

  for k in range(8):
    ones[pl.ds(k * 16, 16)] = one16

  abase = sid * TILE_ROW_BASE
  for off in range(0, TILE_ROW_SPAN, UNIT):
    pltpu.sync_copy(rows.at[0, pl.ds(0, UNIT)],
                    acc.at[pl.ds(abase + off, UNIT)])
  pltpu.sync_copy(zbuf, cnt_sh.at[pl.ds(abase, TILE_ROW_SPAN)])

  plsc.subcore_barrier()

  u0 = (N_UNITS * sid) // NUM_SUBCORES
  u1 = (N_UNITS * (sid + 1)) // NUM_SUBCORES
  n_units = u1 - u0

  def _drain_cnt():
    pltpu.make_async_copy(zbuf.at[pl.ds(0, UNIT)],
                          cnt_sh.at[pl.ds(0, UNIT)], semc).wait()

  def _drain_rows(sem):
    pltpu.make_async_copy(rows.at[0], acc.at[pl.ds(0, UNIT)], sem).wait()

  def _wait_fill(i):
    @pl.when(lax.rem(i, 2) == 0)
    def _():
      _drain_rows(semf0)

    @pl.when(lax.rem(i, 2) == 1)
    def _():
      _drain_rows(semf1)

  def _issue_scatter(i):
    slot = lax.rem(i, 2)
    p = lax.rem(i // BATCH, 2)
    r = lax.rem(i, BATCH)

    pltpu.async_copy(rows.at[slot], acc.at[didxb.at[p, r]], sems,
                     add=True)

    @pl.when(cid == 1)
    def _():
      pltpu.async_copy(ones, cnt_sh.at[didxb.at[p, r]], semc, add=True)

  def _unit(i, carry):
    slot = lax.rem(i, 2)
    p = lax.rem(i // BATCH, 2)
    r = lax.rem(i, BATCH)
    u = u0 + i

    @pl.when(i > 1)
    def _():
      _drain_rows(sems)

      @pl.when(cid == 1)
      def _():
        _drain_cnt()

    @pl.when(r == 0)
    def _():
      pltpu.sync_copy(dst2d.at[pl.ds(u, BATCH)], didxb.at[p])

      @pl.when(cid == 0)
      def _():
        pltpu.sync_copy(src2d.at[pl.ds(u, BATCH)], sidxb.at[p])

    @pl.when(jnp.logical_and(cid == 0, slot == 0))
    def _():
      pltpu.async_copy(nf.at[sidxb.at[p, r]], rows.at[0], semf0)

    @pl.when(jnp.logical_and(cid == 0, slot == 1))
    def _():
      pltpu.async_copy(nf.at[sidxb.at[p, r]], rows.at[1], semf1)

    @pl.when(jnp.logical_and(cid == 1, slot == 0))
    def _():
      pltpu.async_copy(ef.at[pl.ds(u * UNIT, UNIT)], rows.at[0], semf0)

    @pl.when(jnp.logical_and(cid == 1, slot == 1))
    def _():
      pltpu.async_copy(ef.at[pl.ds(u * UNIT, UNIT)], rows.at[1], semf1)

    @pl.when(i > 0)
    def _():
      _wait_fill(i - 1)
      _issue_scatter(i - 1)

    return carry

  lax.fori_loop(0, n_units, _unit, 0)

  _wait_fill(n_units - 1)
  _issue_scatter(n_units - 1)
  _drain_rows(sems)
  _drain_rows(sems)

  @pl.when(cid == 1)
  def _():
    _drain_cnt()
    _drain_cnt()

  plsc.subcore_barrier()

  @pl.when(cid == 0)
  def _out_g():
    pltpu.sync_copy(acc.at[pl.ds(abase, TILE_ROW_SPAN)],
                    g_out.at[pl.ds(abase, TILE_ROW_SPAN)])

  @pl.when(cid == 1)
  def _out_e():
    pltpu.sync_copy(acc.at[pl.ds(abase, TILE_ROW_SPAN)],
                    e_out.at[pl.ds(abase, TILE_ROW_SPAN)])
    pltpu.sync_copy(cnt_sh.at[pl.ds(abase, TILE_ROW_SPAN)], zbuf)
    pltpu.sync_copy(zbuf, cnt_out.at[pl.ds(abase, TILE_ROW_SPAN)])


def _segment_sums(src2d, dst2d, node_feats, edge_feats):
  mesh = plsc.VectorSubcoreMesh(
      core_axis_name="c", subcore_axis_name="s",
      num_cores=NUM_CORES, num_subcores=NUM_SUBCORES)
  f = pl.kernel(
      _sc_body,
      out_type=[
          jax.ShapeDtypeStruct((N_NODES, D), jnp.float32),
          jax.ShapeDtypeStruct((N_NODES, D), jnp.float32),
          jax.ShapeDtypeStruct((N_NODES,), jnp.float32),
      ],
      mesh=mesh,
      scratch_types=[
          pltpu.VMEM((2, BATCH, 128), jnp.int32),
          pltpu.VMEM((2, BATCH, 128), jnp.int32),
          pltpu.VMEM((2, UNIT, D), jnp.float32),
          pltpu.VMEM((128,), jnp.float32),
          pltpu.VMEM((TILE_ROW_SPAN,), jnp.float32),
          pltpu.VMEM_SHARED((N_NODES, D), jnp.float32),
          pltpu.VMEM_SHARED((N_NODES,), jnp.float32),
          pltpu.SemaphoreType.DMA,
          pltpu.SemaphoreType.DMA,
          pltpu.SemaphoreType.DMA,
          pltpu.SemaphoreType.DMA,
      ],
      compiler_params=pltpu.CompilerParams(use_tc_tiling_on_sc=False),
  )
  return f(src2d, dst2d, node_feats, edge_feats)


def _tail_body(g_ref, e_ref, cntp_ref, w1a_ref, w1b_ref, w2_ref,
               b1_ref, b2_ref, out_ref):
  cnt = cntp_ref[...][:, 0]
  ms = (jnp.dot(g_ref[...], w1a_ref[...], preferred_element_type=jnp.float32)
        + jnp.dot(e_ref[...], w1b_ref[...], preferred_element_type=jnp.float32)
        + cnt[:, None] * b1_ref[...])
  nm = ms / jnp.maximum(cnt, 1.0)[:, None]
  sm = jnp.dot(nm, w2_ref[...], preferred_element_type=jnp.float32) + b2_ref[...]
  o = nm + sm
  out_ref[...] = jnp.where(o >= 0, o, o * RRELU_SLOPE)


def _tail(g, e, cntp, w1a_t, w1b_t, w2_t, b1, b2):
  blk = 2000
  grid = (N_NODES // blk,)
  return pl.pallas_call(
      _tail_body,
      grid=grid,
      in_specs=[
          pl.BlockSpec((blk, D), lambda i: (i, 0)),
          pl.BlockSpec((blk, D), lambda i: (i, 0)),
          pl.BlockSpec((blk, 1), lambda i: (i, 0)),
          pl.BlockSpec((D, D), lambda i: (0, 0)),
          pl.BlockSpec((D, D), lambda i: (0, 0)),
          pl.BlockSpec((D, D), lambda i: (0, 0)),
          pl.BlockSpec((1, D), lambda i: (0, 0)),
          pl.BlockSpec((1, D), lambda i: (0, 0)),
      ],
      out_specs=pl.BlockSpec((blk, D), lambda i: (i, 0)),
      out_shape=jax.ShapeDtypeStruct((N_NODES, D), jnp.float32),
  )(g, e, cntp, w1a_t, w1b_t, w2_t, b1, b2)


@jax.jit
def kernel(node_feats, edge_feats, edge_index, W1, b1, W2, b2):
  pad = PAD_ROWS * 128 - N_EDGES
  src2d = jnp.pad(edge_index[0].astype(jnp.int32),
                  (0, pad)).reshape(PAD_ROWS, 128)
  dst2d = jnp.pad(edge_index[1].astype(jnp.int32),
                  (0, pad)).reshape(PAD_ROWS, 128)
  g, e, cnt = _segment_sums(src2d, dst2d, node_feats, edge_feats)
  cntp = cnt.reshape(N_NODES, 1)
  w1a_t = W1[:, :D].T
  w1b_t = W1[:, D:].T
  w2_t = W2.T
  return _tail(g, e, cntp, w1a_t, w1b_t, w2_t,
               b1.reshape(1, D), b2.reshape(1, D))


# speedup vs baseline: 1.0219x; 1.0025x over previous
import functools

import jax
import jax.numpy as jnp
from jax import lax
from jax.experimental import pallas as pl
from jax.experimental.pallas import tpu as pltpu
from jax.experimental.pallas import tpu_sc as plsc

N_NODES = 10000
N_EDGES = 320000
D = 128
RRELU_SLOPE = (1.0 / 8.0 + 1.0 / 3.0) / 2.0

NUM_CORES = 2
NUM_SUBCORES = 16
TILE_ROW_BASE = 624
TILE_ROW_SPAN = 640
UNIT = 128
N_UNITS = N_EDGES // UNIT
BATCH = 16
PAD_ROWS = 2512


def _sc_body(src2d, dst2d, nf, ef, g_out, e_out, cnt_out,
             sidxb, didxb, rows, ones, zbuf, acc, cnt_sh,
             semf0, semf1, sems, semc):
  cid = lax.axis_index("c")
  sid = lax.axis_index("s")

  zero16 = jnp.zeros((16,), jnp.float32)
  one16 = jnp.ones((16,), jnp.float32)

  def _zero_rows(i, carry):
    for k in range(8):
      rows[0, i, pl.ds(k * 16, 16)] = zero16
    return carry
  lax.fori_loop(0, UNIT, _zero_rows, 0)

  def _zero_z(i, carry):
    zbuf[pl.ds(i * 16, 16)] = zero16
    return carry
  lax.fori_loop(0, TILE_ROW_SPAN // 16, _zero_z, 0)

  for k in range(8):
    ones[pl.ds(k * 16, 16)] = one16

  abase = sid * TILE_ROW_BASE
  for off in range(0, TILE_ROW_SPAN, UNIT):
    pltpu.sync_copy(rows.at[0, pl.ds(0, UNIT)],
                    acc.at[pl.ds(abase + off, UNIT)])
  pltpu.sync_copy(zbuf, cnt_sh.at[pl.ds(abase, TILE_ROW_SPAN)])

  plsc.subcore_barrier()

  u0 = (N_UNITS * sid) // NUM_SUBCORES
  u1 = (N_UNITS * (sid + 1)) // NUM_SUBCORES
  n_units = u1 - u0

  def _drain_cnt():
    pltpu.make_async_copy(zbuf.at[pl.ds(0, UNIT)],
                          cnt_sh.at[pl.ds(0, UNIT)], semc).wait()

  def _drain_rows(sem):
    pltpu.make_async_copy(rows.at[0], acc.at[pl.ds(0, UNIT)], sem).wait()

  def _wait_fill(i):
    @pl.when(lax.rem(i, 2) == 0)
    def _():
      _drain_rows(semf0)

    @pl.when(lax.rem(i, 2) == 1)
    def _():
      _drain_rows(semf1)

  def _issue_scatter(i):
    slot = lax.rem(i, 2)
    p = lax.rem(i // BATCH, 2)
    r = lax.rem(i, BATCH)

    pltpu.async_copy(rows.at[slot], acc.at[didxb.at[p, r]], sems,
                     add=True)

    @pl.when(cid == 1)
    def _():
      pltpu.async_copy(ones, cnt_sh.at[didxb.at[p, r]], semc, add=True)

  def _unit(i, carry):
    slot = lax.rem(i, 2)
    p = lax.rem(i // BATCH, 2)
    r = lax.rem(i, BATCH)
    u = u0 + i

    @pl.when(i > 1)
    def _():
      _drain_rows(sems)

      @pl.when(cid == 1)
      def _():
        _drain_cnt()

    @pl.when(r == 0)
    def _():
      pltpu.sync_copy(dst2d.at[pl.ds(u, BATCH)], didxb.at[p])

      @pl.when(cid == 0)
      def _():
        pltpu.sync_copy(src2d.at[pl.ds(u, BATCH)], sidxb.at[p])

    @pl.when(jnp.logical_and(cid == 0, slot == 0))
    def _():
      pltpu.async_copy(nf.at[sidxb.at[p, r]], rows.at[0], semf0)

    @pl.when(jnp.logical_and(cid == 0, slot == 1))
    def _():
      pltpu.async_copy(nf.at[sidxb.at[p, r]], rows.at[1], semf1)

    @pl.when(jnp.logical_and(cid == 1, slot == 0))
    def _():
      pltpu.async_copy(ef.at[pl.ds(u * UNIT, UNIT)], rows.at[0], semf0)

    @pl.when(jnp.logical_and(cid == 1, slot == 1))
    def _():
      pltpu.async_copy(ef.at[pl.ds(u * UNIT, UNIT)], rows.at[1], semf1)

    @pl.when(i > 0)
    def _():
      _wait_fill(i - 1)
      _issue_scatter(i - 1)

    return carry

  lax.fori_loop(0, n_units, _unit, 0)

  _wait_fill(n_units - 1)
  _issue_scatter(n_units - 1)
  _drain_rows(sems)
  _drain_rows(sems)

  @pl.when(cid == 1)
  def _():
    _drain_cnt()
    _drain_cnt()

  plsc.subcore_barrier()

  @pl.when(cid == 0)
  def _out_g():
    pltpu.sync_copy(acc.at[pl.ds(abase, TILE_ROW_SPAN)],
                    g_out.at[pl.ds(abase, TILE_ROW_SPAN)])

  @pl.when(cid == 1)
  def _out_e():
    pltpu.sync_copy(acc.at[pl.ds(abase, TILE_ROW_SPAN)],
                    e_out.at[pl.ds(abase, TILE_ROW_SPAN)])
    pltpu.sync_copy(cnt_sh.at[pl.ds(abase, TILE_ROW_SPAN)], zbuf)
    pltpu.sync_copy(zbuf, cnt_out.at[pl.ds(abase, TILE_ROW_SPAN)])


def _segment_sums(src2d, dst2d, node_feats, edge_feats):
  mesh = plsc.VectorSubcoreMesh(
      core_axis_name="c", subcore_axis_name="s",
      num_cores=NUM_CORES, num_subcores=NUM_SUBCORES)
  f = pl.kernel(
      _sc_body,
      out_type=[
          jax.ShapeDtypeStruct((N_NODES, D), jnp.float32),
          jax.ShapeDtypeStruct((N_NODES, D), jnp.float32),
          jax.ShapeDtypeStruct((N_NODES,), jnp.float32),
      ],
      mesh=mesh,
      scratch_types=[
          pltpu.VMEM((2, BATCH, 128), jnp.int32),
          pltpu.VMEM((2, BATCH, 128), jnp.int32),
          pltpu.VMEM((2, UNIT, D), jnp.float32),
          pltpu.VMEM((128,), jnp.float32),
          pltpu.VMEM((TILE_ROW_SPAN,), jnp.float32),
          pltpu.VMEM_SHARED((N_NODES, D), jnp.float32),
          pltpu.VMEM_SHARED((N_NODES,), jnp.float32),
          pltpu.SemaphoreType.DMA,
          pltpu.SemaphoreType.DMA,
          pltpu.SemaphoreType.DMA,
          pltpu.SemaphoreType.DMA,
      ],
      compiler_params=pltpu.CompilerParams(use_tc_tiling_on_sc=False),
  )
  return f(src2d, dst2d, node_feats, edge_feats)


def _matmul_nt(x, w):
  return lax.dot_general(x, w, (((1,), (1,)), ((), ())),
                         preferred_element_type=jnp.float32)


def _tail_body(g_ref, e_ref, cntp_ref, w1_ref, w2_ref,
               b1_ref, b2_ref, out_ref):
  cnt = cntp_ref[...][:, 0]
  ms = (_matmul_nt(g_ref[...], w1_ref[:, :D])
        + _matmul_nt(e_ref[...], w1_ref[:, D:])
        + cnt[:, None] * b1_ref[...])
  nm = ms / jnp.maximum(cnt, 1.0)[:, None]
  sm = _matmul_nt(nm, w2_ref[...]) + b2_ref[...]
  o = nm + sm
  out_ref[...] = jnp.where(o >= 0, o, o * RRELU_SLOPE)


def _tail(g, e, cntp, w1, w2, b1, b2):
  blk = 2000
  grid = (N_NODES // blk,)
  return pl.pallas_call(
      _tail_body,
      grid=grid,
      in_specs=[
          pl.BlockSpec((blk, D), lambda i: (i, 0)),
          pl.BlockSpec((blk, D), lambda i: (i, 0)),
          pl.BlockSpec((blk, 1), lambda i: (i, 0)),
          pl.BlockSpec((D, 2 * D), lambda i: (0, 0)),
          pl.BlockSpec((D, D), lambda i: (0, 0)),
          pl.BlockSpec((1, D), lambda i: (0, 0)),
          pl.BlockSpec((1, D), lambda i: (0, 0)),
      ],
      out_specs=pl.BlockSpec((blk, D), lambda i: (i, 0)),
      out_shape=jax.ShapeDtypeStruct((N_NODES, D), jnp.float32),
  )(g, e, cntp, w1, w2, b1, b2)


@jax.jit
def kernel(node_feats, edge_feats, edge_index, W1, b1, W2, b2):
  pad = PAD_ROWS * 128 - N_EDGES
  src2d = jnp.pad(edge_index[0].astype(jnp.int32),
                  (0, pad)).reshape(PAD_ROWS, 128)
  dst2d = jnp.pad(edge_index[1].astype(jnp.int32),
                  (0, pad)).reshape(PAD_ROWS, 128)
  g, e, cnt = _segment_sums(src2d, dst2d, node_feats, edge_feats)
  cntp = cnt.reshape(N_NODES, 1)
  return _tail(g, e, cntp, W1, W2,
               b1.reshape(1, D), b2.reshape(1, D))


# speedup vs baseline: 1.0505x; 1.0280x over previous
import functools

import jax
import jax.numpy as jnp
from jax import lax
from jax.experimental import pallas as pl
from jax.experimental.pallas import tpu as pltpu
from jax.experimental.pallas import tpu_sc as plsc

N_NODES = 10000
N_EDGES = 320000
D = 128
RRELU_SLOPE = (1.0 / 8.0 + 1.0 / 3.0) / 2.0

NUM_CORES = 2
NUM_SUBCORES = 16
TILE_ROW_BASE = 624
TILE_ROW_SPAN = 640
UNIT = 128
N_UNITS = N_EDGES // UNIT
BATCH = 32
PAD_ROWS = 2512


def _sc_body(src2d, dst2d, nf, ef, g_out, e_out, cnt_out,
             sidxb, didxb, rows, ones, zbuf, acc, cnt_sh,
             semf0, semf1, sems, semc):
  cid = lax.axis_index("c")
  sid = lax.axis_index("s")

  zero16 = jnp.zeros((16,), jnp.float32)
  one16 = jnp.ones((16,), jnp.float32)

  def _zero_rows(i, carry):
    for k in range(8):
      rows[0, i, pl.ds(k * 16, 16)] = zero16
    return carry
  lax.fori_loop(0, UNIT, _zero_rows, 0)

  def _zero_z(i, carry):
    zbuf[pl.ds(i * 16, 16)] = zero16
    return carry
  lax.fori_loop(0, TILE_ROW_SPAN // 16, _zero_z, 0)

  for k in range(8):
    ones[pl.ds(k * 16, 16)] = one16

  abase = sid * TILE_ROW_BASE
  for off in range(0, TILE_ROW_SPAN, UNIT):
    pltpu.sync_copy(rows.at[0, pl.ds(0, UNIT)],
                    acc.at[pl.ds(abase + off, UNIT)])
  pltpu.sync_copy(zbuf, cnt_sh.at[pl.ds(abase, TILE_ROW_SPAN)])

  plsc.subcore_barrier()

  u0 = (N_UNITS * sid) // NUM_SUBCORES
  u1 = (N_UNITS * (sid + 1)) // NUM_SUBCORES
  n_units = u1 - u0

  def _drain_cnt():
    pltpu.make_async_copy(zbuf.at[pl.ds(0, UNIT)],
                          cnt_sh.at[pl.ds(0, UNIT)], semc).wait()

  def _drain_rows(sem):
    pltpu.make_async_copy(rows.at[0], acc.at[pl.ds(0, UNIT)], sem).wait()

  def _wait_fill(i):
    @pl.when(lax.rem(i, 2) == 0)
    def _():
      _drain_rows(semf0)

    @pl.when(lax.rem(i, 2) == 1)
    def _():
      _drain_rows(semf1)

  def _issue_scatter(i):
    slot = lax.rem(i, 2)
    p = lax.rem(i // BATCH, 2)
    r = lax.rem(i, BATCH)

    pltpu.async_copy(rows.at[slot], acc.at[didxb.at[p, r]], sems,
                     add=True)

    @pl.when(cid == 1)
    def _():
      pltpu.async_copy(ones, cnt_sh.at[didxb.at[p, r]], semc, add=True)

  def _unit(i, carry):
    slot = lax.rem(i, 2)
    p = lax.rem(i // BATCH, 2)
    r = lax.rem(i, BATCH)
    u = u0 + i

    @pl.when(i > 1)
    def _():
      _drain_rows(sems)

      @pl.when(cid == 1)
      def _():
        _drain_cnt()

    @pl.when(r == 0)
    def _():
      pltpu.sync_copy(dst2d.at[pl.ds(u, BATCH)], didxb.at[p])

      @pl.when(cid == 0)
      def _():
        pltpu.sync_copy(src2d.at[pl.ds(u, BATCH)], sidxb.at[p])

    @pl.when(jnp.logical_and(cid == 0, slot == 0))
    def _():
      pltpu.async_copy(nf.at[sidxb.at[p, r]], rows.at[0], semf0)

    @pl.when(jnp.logical_and(cid == 0, slot == 1))
    def _():
      pltpu.async_copy(nf.at[sidxb.at[p, r]], rows.at[1], semf1)

    @pl.when(jnp.logical_and(cid == 1, slot == 0))
    def _():
      pltpu.async_copy(ef.at[pl.ds(u * UNIT, UNIT)], rows.at[0], semf0)

    @pl.when(jnp.logical_and(cid == 1, slot == 1))
    def _():
      pltpu.async_copy(ef.at[pl.ds(u * UNIT, UNIT)], rows.at[1], semf1)

    @pl.when(i > 0)
    def _():
      _wait_fill(i - 1)
      _issue_scatter(i - 1)

    return carry

  lax.fori_loop(0, n_units, _unit, 0)

  _wait_fill(n_units - 1)
  _issue_scatter(n_units - 1)
  _drain_rows(sems)
  _drain_rows(sems)

  @pl.when(cid == 1)
  def _():
    _drain_cnt()
    _drain_cnt()

  plsc.subcore_barrier()

  @pl.when(cid == 0)
  def _out_g():
    pltpu.sync_copy(acc.at[pl.ds(abase, TILE_ROW_SPAN)],
                    g_out.at[pl.ds(abase, TILE_ROW_SPAN)])

  @pl.when(cid == 1)
  def _out_e():
    pltpu.sync_copy(acc.at[pl.ds(abase, TILE_ROW_SPAN)],
                    e_out.at[pl.ds(abase, TILE_ROW_SPAN)])
    pltpu.sync_copy(cnt_sh.at[pl.ds(abase, TILE_ROW_SPAN)], zbuf)
    pltpu.sync_copy(zbuf, cnt_out.at[pl.ds(abase, TILE_ROW_SPAN)])


def _segment_sums(src2d, dst2d, node_feats, edge_feats):
  mesh = plsc.VectorSubcoreMesh(
      core_axis_name="c", subcore_axis_name="s",
      num_cores=NUM_CORES, num_subcores=NUM_SUBCORES)
  f = pl.kernel(
      _sc_body,
      out_type=[
          jax.ShapeDtypeStruct((N_NODES, D), jnp.float32),
          jax.ShapeDtypeStruct((N_NODES, D), jnp.float32),
          jax.ShapeDtypeStruct((N_NODES,), jnp.float32),
      ],
      mesh=mesh,
      scratch_types=[
          pltpu.VMEM((2, BATCH, 128), jnp.int32),
          pltpu.VMEM((2, BATCH, 128), jnp.int32),
          pltpu.VMEM((2, UNIT, D), jnp.float32),
          pltpu.VMEM((128,), jnp.float32),
          pltpu.VMEM((TILE_ROW_SPAN,), jnp.float32),
          pltpu.VMEM_SHARED((N_NODES, D), jnp.float32),
          pltpu.VMEM_SHARED((N_NODES,), jnp.float32),
          pltpu.SemaphoreType.DMA,
          pltpu.SemaphoreType.DMA,
          pltpu.SemaphoreType.DMA,
          pltpu.SemaphoreType.DMA,
      ],
      compiler_params=pltpu.CompilerParams(use_tc_tiling_on_sc=False),
  )
  return f(src2d, dst2d, node_feats, edge_feats)


def _matmul_nt(x, w):
  return lax.dot_general(x, w, (((1,), (1,)), ((), ())),
                         preferred_element_type=jnp.float32)


def _tail_body(g_ref, e_ref, cntp_ref, w1_ref, w2_ref,
               b1_ref, b2_ref, out_ref):
  cnt = cntp_ref[...][:, 0]
  ms = (_matmul_nt(g_ref[...], w1_ref[:, :D])
        + _matmul_nt(e_ref[...], w1_ref[:, D:])
        + cnt[:, None] * b1_ref[...])
  nm = ms / jnp.maximum(cnt, 1.0)[:, None]
  sm = _matmul_nt(nm, w2_ref[...]) + b2_ref[...]
  o = nm + sm
  out_ref[...] = jnp.where(o >= 0, o, o * RRELU_SLOPE)


def _tail(g, e, cntp, w1, w2, b1, b2):
  blk = 2000
  grid = (N_NODES // blk,)
  return pl.pallas_call(
      _tail_body,
      grid=grid,
      in_specs=[
          pl.BlockSpec((blk, D), lambda i: (i, 0)),
          pl.BlockSpec((blk, D), lambda i: (i, 0)),
          pl.BlockSpec((blk, 1), lambda i: (i, 0)),
          pl.BlockSpec((D, 2 * D), lambda i: (0, 0)),
          pl.BlockSpec((D, D), lambda i: (0, 0)),
          pl.BlockSpec((1, D), lambda i: (0, 0)),
          pl.BlockSpec((1, D), lambda i: (0, 0)),
      ],
      out_specs=pl.BlockSpec((blk, D), lambda i: (i, 0)),
      out_shape=jax.ShapeDtypeStruct((N_NODES, D), jnp.float32),
  )(g, e, cntp, w1, w2, b1, b2)


@jax.jit
def kernel(node_feats, edge_feats, edge_index, W1, b1, W2, b2):
  pad = PAD_ROWS * 128 - N_EDGES
  src2d = jnp.pad(edge_index[0].astype(jnp.int32),
                  (0, pad)).reshape(PAD_ROWS, 128)
  dst2d = jnp.pad(edge_index[1].astype(jnp.int32),
                  (0, pad)).reshape(PAD_ROWS, 128)
  g, e, cnt = _segment_sums(src2d, dst2d, node_feats, edge_feats)
  cntp = cnt.reshape(N_NODES, 1)
  return _tail(g, e, cntp, W1, W2,
               b1.reshape(1, D), b2.reshape(1, D))


# speedup vs baseline: 1.0599x; 1.0090x over previous
import functools

import jax
import jax.numpy as jnp
from jax import lax
from jax.experimental import pallas as pl
from jax.experimental.pallas import tpu as pltpu
from jax.experimental.pallas import tpu_sc as plsc

N_NODES = 10000
N_EDGES = 320000
D = 128
RRELU_SLOPE = (1.0 / 8.0 + 1.0 / 3.0) / 2.0

NUM_CORES = 2
NUM_SUBCORES = 16
TILE_ROW_BASE = 624
TILE_ROW_SPAN = 640
UNIT = 128
N_UNITS = N_EDGES // UNIT
BATCH = 16
NPAR = 3
PAD_ROWS = 2528


def _sc_body(src2d, dst2d, nf, ef, g_out, e_out, cnt_out,
             sidxb, didxb, rows, ones, zbuf, acc, cnt_sh,
             semf0, semf1, sems, semc, semi):
  cid = lax.axis_index("c")
  sid = lax.axis_index("s")

  zero16 = jnp.zeros((16,), jnp.float32)
  one16 = jnp.ones((16,), jnp.float32)

  def _zero_rows(i, carry):
    for k in range(8):
      rows[0, i, pl.ds(k * 16, 16)] = zero16
    return carry
  lax.fori_loop(0, UNIT, _zero_rows, 0)

  def _zero_z(i, carry):
    zbuf[pl.ds(i * 16, 16)] = zero16
    return carry
  lax.fori_loop(0, TILE_ROW_SPAN // 16, _zero_z, 0)

  for k in range(8):
    ones[pl.ds(k * 16, 16)] = one16

  abase = sid * TILE_ROW_BASE
  for off in range(0, TILE_ROW_SPAN, UNIT):
    pltpu.sync_copy(rows.at[0, pl.ds(0, UNIT)],
                    acc.at[pl.ds(abase + off, UNIT)])
  pltpu.sync_copy(zbuf, cnt_sh.at[pl.ds(abase, TILE_ROW_SPAN)])

  plsc.subcore_barrier()

  u0 = (N_UNITS * sid) // NUM_SUBCORES
  u1 = (N_UNITS * (sid + 1)) // NUM_SUBCORES
  n_units = u1 - u0

  def _drain_cnt():
    pltpu.make_async_copy(zbuf.at[pl.ds(0, UNIT)],
                          cnt_sh.at[pl.ds(0, UNIT)], semc).wait()

  def _drain_rows(sem):
    pltpu.make_async_copy(rows.at[0], acc.at[pl.ds(0, UNIT)], sem).wait()

  def _wait_fill(i):
    @pl.when(lax.rem(i, 2) == 0)
    def _():
      _drain_rows(semf0)

    @pl.when(lax.rem(i, 2) == 1)
    def _():
      _drain_rows(semf1)

  def _issue_scatter(i):
    slot = lax.rem(i, 2)
    p = lax.rem(i // BATCH, NPAR)
    r = lax.rem(i, BATCH)

    pltpu.async_copy(rows.at[slot], acc.at[didxb.at[p, r]], sems,
                     add=True)

    @pl.when(cid == 1)
    def _():
      pltpu.async_copy(ones, cnt_sh.at[didxb.at[p, r]], semc, add=True)

  def _prefetch(b):
    pn = lax.rem(b, NPAR)
    ub = u0 + b * BATCH
    pltpu.async_copy(dst2d.at[pl.ds(ub, BATCH)], didxb.at[pn], semi)

    @pl.when(cid == 0)
    def _():
      pltpu.async_copy(src2d.at[pl.ds(ub, BATCH)], sidxb.at[pn], semi)

  def _wait_prefetch():
    pltpu.make_async_copy(dst2d.at[pl.ds(0, BATCH)], didxb.at[0],
                          semi).wait()

    @pl.when(cid == 0)
    def _():
      pltpu.make_async_copy(src2d.at[pl.ds(0, BATCH)], sidxb.at[0],
                            semi).wait()

  pltpu.sync_copy(dst2d.at[pl.ds(u0, BATCH)], didxb.at[0])

  @pl.when(cid == 0)
  def _():
    pltpu.sync_copy(src2d.at[pl.ds(u0, BATCH)], sidxb.at[0])

  _prefetch(1)

  def _unit(i, carry):
    slot = lax.rem(i, 2)
    p = lax.rem(i // BATCH, NPAR)
    r = lax.rem(i, BATCH)
    u = u0 + i

    @pl.when(i > 1)
    def _():
      _drain_rows(sems)

      @pl.when(cid == 1)
      def _():
        _drain_cnt()

    @pl.when(jnp.logical_and(r == 0, i > 0))
    def _():
      _wait_prefetch()
      _prefetch(i // BATCH + 1)

    @pl.when(jnp.logical_and(cid == 0, slot == 0))
    def _():
      pltpu.async_copy(nf.at[sidxb.at[p, r]], rows.at[0], semf0)

    @pl.when(jnp.logical_and(cid == 0, slot == 1))
    def _():
      pltpu.async_copy(nf.at[sidxb.at[p, r]], rows.at[1], semf1)

    @pl.when(jnp.logical_and(cid == 1, slot == 0))
    def _():
      pltpu.async_copy(ef.at[pl.ds(u * UNIT, UNIT)], rows.at[0], semf0)

    @pl.when(jnp.logical_and(cid == 1, slot == 1))
    def _():
      pltpu.async_copy(ef.at[pl.ds(u * UNIT, UNIT)], rows.at[1], semf1)

    @pl.when(i > 0)
    def _():
      _wait_fill(i - 1)
      _issue_scatter(i - 1)

    return carry

  lax.fori_loop(0, n_units, _unit, 0)

  _wait_fill(n_units - 1)
  _issue_scatter(n_units - 1)
  _drain_rows(sems)
  _drain_rows(sems)
  _wait_prefetch()

  @pl.when(cid == 1)
  def _():
    _drain_cnt()
    _drain_cnt()

  plsc.subcore_barrier()

  @pl.when(cid == 0)
  def _out_g():
    pltpu.sync_copy(acc.at[pl.ds(abase, TILE_ROW_SPAN)],
                    g_out.at[pl.ds(abase, TILE_ROW_SPAN)])

  @pl.when(cid == 1)
  def _out_e():
    pltpu.sync_copy(acc.at[pl.ds(abase, TILE_ROW_SPAN)],
                    e_out.at[pl.ds(abase, TILE_ROW_SPAN)])
    pltpu.sync_copy(cnt_sh.at[pl.ds(abase, TILE_ROW_SPAN)], zbuf)
    pltpu.sync_copy(zbuf, cnt_out.at[pl.ds(abase, TILE_ROW_SPAN)])


def _segment_sums(src2d, dst2d, node_feats, edge_feats):
  mesh = plsc.VectorSubcoreMesh(
      core_axis_name="c", subcore_axis_name="s",
      num_cores=NUM_CORES, num_subcores=NUM_SUBCORES)
  f = pl.kernel(
      _sc_body,
      out_type=[
          jax.ShapeDtypeStruct((N_NODES, D), jnp.float32),
          jax.ShapeDtypeStruct((N_NODES, D), jnp.float32),
          jax.ShapeDtypeStruct((N_NODES,), jnp.float32),
      ],
      mesh=mesh,
      scratch_types=[
          pltpu.VMEM((NPAR, BATCH, 128), jnp.int32),
          pltpu.VMEM((NPAR, BATCH, 128), jnp.int32),
          pltpu.VMEM((2, UNIT, D), jnp.float32),
          pltpu.VMEM((128,), jnp.float32),
          pltpu.VMEM((TILE_ROW_SPAN,), jnp.float32),
          pltpu.VMEM_SHARED((N_NODES, D), jnp.float32),
          pltpu.VMEM_SHARED((N_NODES,), jnp.float32),
          pltpu.SemaphoreType.DMA,
          pltpu.SemaphoreType.DMA,
          pltpu.SemaphoreType.DMA,
          pltpu.SemaphoreType.DMA,
          pltpu.SemaphoreType.DMA,
      ],
      compiler_params=pltpu.CompilerParams(use_tc_tiling_on_sc=False),
  )
  return f(src2d, dst2d, node_feats, edge_feats)


def _matmul_nt(x, w):
  return lax.dot_general(x, w, (((1,), (1,)), ((), ())),
                         preferred_element_type=jnp.float32)


def _tail_body(g_ref, e_ref, cntp_ref, w1_ref, w2_ref,
               b1_ref, b2_ref, out_ref):
  cnt = cntp_ref[...][:, 0]
  ms = (_matmul_nt(g_ref[...], w1_ref[:, :D])
        + _matmul_nt(e_ref[...], w1_ref[:, D:])
        + cnt[:, None] * b1_ref[...])
  nm = ms / jnp.maximum(cnt, 1.0)[:, None]
  sm = _matmul_nt(nm, w2_ref[...]) + b2_ref[...]
  o = nm + sm
  out_ref[...] = jnp.where(o >= 0, o, o * RRELU_SLOPE)


def _tail(g, e, cntp, w1, w2, b1, b2):
  blk = 2000
  grid = (N_NODES // blk,)
  return pl.pallas_call(
      _tail_body,
      grid=grid,
      in_specs=[
          pl.BlockSpec((blk, D), lambda i: (i, 0)),
          pl.BlockSpec((blk, D), lambda i: (i, 0)),
          pl.BlockSpec((blk, 1), lambda i: (i, 0)),
          pl.BlockSpec((D, 2 * D), lambda i: (0, 0)),
          pl.BlockSpec((D, D), lambda i: (0, 0)),
          pl.BlockSpec((1, D), lambda i: (0, 0)),
          pl.BlockSpec((1, D), lambda i: (0, 0)),
      ],
      out_specs=pl.BlockSpec((blk, D), lambda i: (i, 0)),
      out_shape=jax.ShapeDtypeStruct((N_NODES, D), jnp.float32),
  )(g, e, cntp, w1, w2, b1, b2)


@jax.jit
def kernel(node_feats, edge_feats, edge_index, W1, b1, W2, b2):
  pad = PAD_ROWS * 128 - N_EDGES
  src2d = jnp.pad(edge_index[0].astype(jnp.int32),
                  (0, pad)).reshape(PAD_ROWS, 128)
  dst2d = jnp.pad(edge_index[1].astype(jnp.int32),
                  (0, pad)).reshape(PAD_ROWS, 128)
  g, e, cnt = _segment_sums(src2d, dst2d, node_feats, edge_feats)
  cntp = cnt.reshape(N_NODES, 1)
  return _tail(g, e, cntp, W1, W2,
               b1.reshape(1, D), b2.reshape(1, D))


# speedup vs baseline: 1.0621x; 1.0020x over previous
import jax
import jax.numpy as jnp
from jax import lax
from jax.experimental import pallas as pl
from jax.experimental.pallas import tpu as pltpu
from jax.experimental.pallas import tpu_sc as plsc

N_NODES = 10000
N_EDGES = 320000
D = 128
RRELU_SLOPE = (1.0 / 8.0 + 1.0 / 3.0) / 2.0

NUM_CORES = 2
NUM_SUBCORES = 16
TILE_ROW_BASE = 624
TILE_ROW_SPAN = 640
UNIT = 128
N_UNITS = N_EDGES // UNIT
BATCH = 16
NPAR = 3
PAD_ROWS = 2528


def _sc_body(src2d, dst2d, nf, ef, g_out, e_out, cnt_out,
             sidxb, didxb, rows, ones, zbuf, acc, cnt_sh,
             semf0, semf1, sems, semc, semi):
  cid = lax.axis_index("c")
  sid = lax.axis_index("s")

  zero16 = jnp.zeros((16,), jnp.float32)
  one16 = jnp.ones((16,), jnp.float32)

  def _zero_rows(i, carry):
    for k in range(8):
      rows[0, i, pl.ds(k * 16, 16)] = zero16
    return carry
  lax.fori_loop(0, UNIT, _zero_rows, 0)

  def _zero_z(i, carry):
    zbuf[pl.ds(i * 16, 16)] = zero16
    return carry
  lax.fori_loop(0, TILE_ROW_SPAN // 16, _zero_z, 0)

  for k in range(8):
    ones[pl.ds(k * 16, 16)] = one16

  abase = sid * TILE_ROW_BASE
  for off in range(0, TILE_ROW_SPAN, UNIT):
    pltpu.sync_copy(rows.at[0, pl.ds(0, UNIT)],
                    acc.at[pl.ds(abase + off, UNIT)])
  pltpu.sync_copy(zbuf, cnt_sh.at[pl.ds(abase, TILE_ROW_SPAN)])

  plsc.subcore_barrier()

  u0 = (N_UNITS * sid) // NUM_SUBCORES
  u1 = (N_UNITS * (sid + 1)) // NUM_SUBCORES
  n_units = u1 - u0

  def _drain_cnt():
    pltpu.make_async_copy(zbuf.at[pl.ds(0, UNIT)],
                          cnt_sh.at[pl.ds(0, UNIT)], semc).wait()

  def _drain_rows(sem):
    pltpu.make_async_copy(rows.at[0], acc.at[pl.ds(0, UNIT)], sem).wait()

  def _wait_fill(i):
    @pl.when(lax.rem(i, 2) == 0)
    def _():
      _drain_rows(semf0)

    @pl.when(lax.rem(i, 2) == 1)
    def _():
      _drain_rows(semf1)

  def _issue_scatter(i):
    slot = lax.rem(i, 2)
    p = lax.rem(i // BATCH, NPAR)
    r = lax.rem(i, BATCH)

    pltpu.async_copy(rows.at[slot], acc.at[didxb.at[p, r]], sems,
                     add=True)

    @pl.when(cid == 1)
    def _():
      pltpu.async_copy(ones, cnt_sh.at[didxb.at[p, r]], semc, add=True)

  def _prefetch(b):
    pn = lax.rem(b, NPAR)
    ub = u0 + b * BATCH
    pltpu.async_copy(dst2d.at[pl.ds(ub, BATCH)], didxb.at[pn], semi)

    @pl.when(cid == 0)
    def _():
      pltpu.async_copy(src2d.at[pl.ds(ub, BATCH)], sidxb.at[pn], semi)

  def _wait_prefetch():
    pltpu.make_async_copy(dst2d.at[pl.ds(0, BATCH)], didxb.at[0],
                          semi).wait()

    @pl.when(cid == 0)
    def _():
      pltpu.make_async_copy(src2d.at[pl.ds(0, BATCH)], sidxb.at[0],
                            semi).wait()

  pltpu.sync_copy(dst2d.at[pl.ds(u0, BATCH)], didxb.at[0])

  @pl.when(cid == 0)
  def _():
    pltpu.sync_copy(src2d.at[pl.ds(u0, BATCH)], sidxb.at[0])

  _prefetch(1)

  def _unit(i, carry):
    slot = lax.rem(i, 2)
    p = lax.rem(i // BATCH, NPAR)
    r = lax.rem(i, BATCH)
    u = u0 + i

    @pl.when(i > 1)
    def _():
      _drain_rows(sems)

      @pl.when(cid == 1)
      def _():
        _drain_cnt()

    @pl.when(jnp.logical_and(r == 0, i > 0))
    def _():
      _wait_prefetch()
      _prefetch(i // BATCH + 1)

    @pl.when(jnp.logical_and(cid == 0, slot == 0))
    def _():
      pltpu.async_copy(nf.at[sidxb.at[p, r]], rows.at[0], semf0)

    @pl.when(jnp.logical_and(cid == 0, slot == 1))
    def _():
      pltpu.async_copy(nf.at[sidxb.at[p, r]], rows.at[1], semf1)

    @pl.when(jnp.logical_and(cid == 1, slot == 0))
    def _():
      pltpu.async_copy(ef.at[pl.ds(u * UNIT, UNIT)], rows.at[0], semf0)

    @pl.when(jnp.logical_and(cid == 1, slot == 1))
    def _():
      pltpu.async_copy(ef.at[pl.ds(u * UNIT, UNIT)], rows.at[1], semf1)

    @pl.when(i > 0)
    def _():
      _wait_fill(i - 1)
      _issue_scatter(i - 1)

    return carry

  lax.fori_loop(0, n_units, _unit, 0)

  _wait_fill(n_units - 1)
  _issue_scatter(n_units - 1)
  _drain_rows(sems)
  _drain_rows(sems)
  _wait_prefetch()

  @pl.when(cid == 1)
  def _():
    _drain_cnt()
    _drain_cnt()

  plsc.subcore_barrier()

  @pl.when(cid == 0)
  def _out_g():
    pltpu.sync_copy(acc.at[pl.ds(abase, TILE_ROW_SPAN)],
                    g_out.at[pl.ds(abase, TILE_ROW_SPAN)])

  @pl.when(cid == 1)
  def _out_e():
    pltpu.sync_copy(acc.at[pl.ds(abase, TILE_ROW_SPAN)],
                    e_out.at[pl.ds(abase, TILE_ROW_SPAN)])
    pltpu.sync_copy(cnt_sh.at[pl.ds(abase, TILE_ROW_SPAN)], zbuf)
    pltpu.sync_copy(zbuf, cnt_out.at[pl.ds(abase, TILE_ROW_SPAN)])


def _segment_sums(src2d, dst2d, node_feats, edge_feats):
  mesh = plsc.VectorSubcoreMesh(
      core_axis_name="c", subcore_axis_name="s",
      num_cores=NUM_CORES, num_subcores=NUM_SUBCORES)
  f = pl.kernel(
      _sc_body,
      out_type=[
          jax.ShapeDtypeStruct((N_NODES, D), jnp.float32),
          jax.ShapeDtypeStruct((N_NODES, D), jnp.float32),
          jax.ShapeDtypeStruct((N_NODES,), jnp.float32),
      ],
      mesh=mesh,
      scratch_types=[
          pltpu.VMEM((NPAR, BATCH, 128), jnp.int32),
          pltpu.VMEM((NPAR, BATCH, 128), jnp.int32),
          pltpu.VMEM((2, UNIT, D), jnp.float32),
          pltpu.VMEM((128,), jnp.float32),
          pltpu.VMEM((TILE_ROW_SPAN,), jnp.float32),
          pltpu.VMEM_SHARED((N_NODES, D), jnp.float32),
          pltpu.VMEM_SHARED((N_NODES,), jnp.float32),
          pltpu.SemaphoreType.DMA,
          pltpu.SemaphoreType.DMA,
          pltpu.SemaphoreType.DMA,
          pltpu.SemaphoreType.DMA,
          pltpu.SemaphoreType.DMA,
      ],
      compiler_params=pltpu.CompilerParams(use_tc_tiling_on_sc=False),
  )
  return f(src2d, dst2d, node_feats, edge_feats)


def _matmul_nt(x, w):
  return lax.dot_general(x, w, (((1,), (1,)), ((), ())),
                         preferred_element_type=jnp.float32)


def _tail_body(g_ref, e_ref, cntp_ref, w1_ref, w2_ref,
               b1_ref, b2_ref, out_ref):
  cnt = cntp_ref[...][:, 0]
  ms = (_matmul_nt(g_ref[...], w1_ref[:, :D])
        + _matmul_nt(e_ref[...], w1_ref[:, D:])
        + cnt[:, None] * b1_ref[...])
  nm = ms / jnp.maximum(cnt, 1.0)[:, None]
  sm = _matmul_nt(nm, w2_ref[...]) + b2_ref[...]
  o = nm + sm
  out_ref[...] = jnp.where(o >= 0, o, o * RRELU_SLOPE)


def _tail(g, e, cntp, w1, w2, b1, b2):
  blk = 2000
  grid = (N_NODES // blk,)
  return pl.pallas_call(
      _tail_body,
      grid=grid,
      in_specs=[
          pl.BlockSpec((blk, D), lambda i: (i, 0)),
          pl.BlockSpec((blk, D), lambda i: (i, 0)),
          pl.BlockSpec((blk, 1), lambda i: (i, 0)),
          pl.BlockSpec((D, 2 * D), lambda i: (0, 0)),
          pl.BlockSpec((D, D), lambda i: (0, 0)),
          pl.BlockSpec((1, D), lambda i: (0, 0)),
          pl.BlockSpec((1, D), lambda i: (0, 0)),
      ],
      out_specs=pl.BlockSpec((blk, D), lambda i: (i, 0)),
      out_shape=jax.ShapeDtypeStruct((N_NODES, D), jnp.float32),
  )(g, e, cntp, w1, w2, b1, b2)


@jax.jit
def kernel(node_feats, edge_feats, edge_index, W1, b1, W2, b2):
  pad = PAD_ROWS * 128 - N_EDGES
  src2d = jnp.pad(edge_index[0].astype(jnp.int32),
                  (0, pad)).reshape(PAD_ROWS, 128)
  dst2d = jnp.pad(edge_index[1].astype(jnp.int32),
                  (0, pad)).reshape(PAD_ROWS, 128)
  g, e, cnt = _segment_sums(src2d, dst2d, node_feats, edge_feats)
  cntp = cnt.reshape(N_NODES, 1)
  return _tail(g, e, cntp, W1, W2,
               b1.reshape(1, D), b2.reshape(1, D))
